# Initial kernel scaffold; baseline (speedup 1.0000x reference)
import jax
import jax.numpy as jnp
from jax.experimental import pallas as pl


def kernel(x_omic, emb_var, emb_vc, emb_func, W, b):
    raise NotImplementedError("write your pallas kernel here")


# speedup vs baseline: 6.3875x; 6.3875x over previous
import functools

import jax
import jax.numpy as jnp
from jax import lax
from jax.experimental import pallas as pl
from jax.experimental.pallas import tpu as pltpu
from jax.experimental.pallas import tpu_sc as plsc

_CHUNK = 128
_BLK = 512
_D_OUT = 256


def _sc_gather(var_id, vc_id, f_ids_t, emb_var, emb_vc, emb_func):
    info = plsc.get_sparse_core_info()
    nc, ns = info.num_cores, info.num_subcores
    nw = nc * ns
    n_tok = var_id.shape[0]
    per_w = n_tok // nw
    n_chunks = per_w // _CHUNK

    mesh = plsc.VectorSubcoreMesh(core_axis_name="c", subcore_axis_name="s")

    @functools.partial(
        pl.kernel,
        mesh=mesh,
        compiler_params=pltpu.CompilerParams(use_tc_tiling_on_sc=False),
        out_type=[
            jax.ShapeDtypeStruct((n_tok, 128), jnp.float32),
            jax.ShapeDtypeStruct((n_tok, 32), jnp.float32),
            jax.ShapeDtypeStruct((n_tok, 32), jnp.float32),
        ],
        scratch_types=[
            pltpu.VMEM((_CHUNK,), jnp.int32),
            pltpu.VMEM((_CHUNK,), jnp.int32),
            pltpu.VMEM((6, _CHUNK), jnp.int32),
            pltpu.VMEM((_CHUNK, 128), jnp.float32),
            pltpu.VMEM((_CHUNK, 32), jnp.float32),
            pltpu.VMEM((6, _CHUNK, 32), jnp.float32),
            pltpu.VMEM((_CHUNK, 32), jnp.float32),
            pltpu.SemaphoreType.DMA,
        ],
    )
    def k(var_id_h, vc_id_h, fids_h, table_h, vc_tab_h, func_tab_h,
          hvar_h, hvc_h, hfm_h,
          vidx, vcidx, fidx, vrows, vcrows, frows, fmean, sem):
        wid = lax.axis_index("s") * nc + lax.axis_index("c")

        def chunk_body(g, carry):
            base = wid * per_w + g * _CHUNK
            pltpu.sync_copy(var_id_h.at[pl.ds(base, _CHUNK)], vidx)
            pltpu.sync_copy(vc_id_h.at[pl.ds(base, _CHUNK)], vcidx)
            pltpu.sync_copy(fids_h.at[:, pl.ds(base, _CHUNK)], fidx)
            copies = [
                pltpu.async_copy(table_h.at[vidx], vrows, sem),
                pltpu.async_copy(vc_tab_h.at[vcidx], vcrows, sem),
            ]
            for j in range(6):
                copies.append(
                    pltpu.async_copy(func_tab_h.at[fidx.at[j]], frows.at[j], sem))
            for c in copies:
                c.wait()

            def pool_body(t, carry2):
                for half in range(2):
                    sl = pl.ds(half * 16, 16)
                    acc = frows[0, t, sl]
                    for j in range(1, 6):
                        acc = acc + frows[j, t, sl]
                    fmean[t, sl] = acc * (1.0 / 6.0)
                return carry2

            lax.fori_loop(0, _CHUNK, pool_body, 0)

            pltpu.sync_copy(vrows, hvar_h.at[pl.ds(base, _CHUNK)])
            pltpu.sync_copy(vcrows, hvc_h.at[pl.ds(base, _CHUNK)])
            pltpu.sync_copy(fmean, hfm_h.at[pl.ds(base, _CHUNK)])
            return carry

        lax.fori_loop(0, n_chunks, chunk_body, 0)

    return k(var_id, vc_id, f_ids_t, emb_var, emb_vc, emb_func)


def _tc_body(hv_ref, hvc_ref, hfm_ref, vaf_ref, w_ref, b_ref, out_ref):
    wv = w_ref[0:128, :]
    wvc = w_ref[128:160, :]
    wfm = w_ref[160:192, :]
    wvaf = w_ref[192:193, :]
    acc = jnp.dot(hv_ref[...], wv, preferred_element_type=jnp.float32)
    acc = acc + jnp.dot(hvc_ref[...], wvc, preferred_element_type=jnp.float32)
    acc = acc + jnp.dot(hfm_ref[...], wfm, preferred_element_type=jnp.float32)
    acc = acc + vaf_ref[...] * wvaf
    acc = acc + b_ref[...]
    out_ref[...] = jnp.where(acc > 0.0, acc,
                             jnp.exp(jnp.minimum(acc, 0.0)) - 1.0)


def _tc_project(hvar, hvc, hfm, vaf, w, b2):
    n_tok = hvar.shape[0]
    grid = (n_tok // _BLK,)
    return pl.pallas_call(
        _tc_body,
        grid=grid,
        in_specs=[
            pl.BlockSpec((_BLK, 128), lambda i: (i, 0)),
            pl.BlockSpec((_BLK, 32), lambda i: (i, 0)),
            pl.BlockSpec((_BLK, 32), lambda i: (i, 0)),
            pl.BlockSpec((_BLK, 1), lambda i: (i, 0)),
            pl.BlockSpec((193, _D_OUT), lambda i: (0, 0)),
            pl.BlockSpec((1, _D_OUT), lambda i: (0, 0)),
        ],
        out_specs=pl.BlockSpec((_BLK, _D_OUT), lambda i: (i, 0)),
        out_shape=jax.ShapeDtypeStruct((n_tok, _D_OUT), jnp.float32),
        compiler_params=pltpu.CompilerParams(
            dimension_semantics=("arbitrary",)),
    )(hvar, hvc, hfm, vaf, w, b2)


def kernel(x_omic, emb_var, emb_vc, emb_func, W, b):
    bsz, seq, _ = x_omic.shape
    n_tok = bsz * seq
    xf = x_omic.reshape(n_tok, 9)
    var_id = xf[:, 0].astype(jnp.int32)
    vc_id = xf[:, 1].astype(jnp.int32)
    f_ids_t = xf[:, 2:8].astype(jnp.int32).T
    vaf = xf[:, 8:9]
    hvar, hvc, hfm = _sc_gather(var_id, vc_id, f_ids_t,
                                emb_var, emb_vc, emb_func)
    out = _tc_project(hvar, hvc, hfm, vaf, W, b.reshape(1, -1))
    return out.reshape(bsz, seq, _D_OUT)
